# baseline (speedup 1.0000x reference)
import functools

import jax
import jax.numpy as jnp
from jax import lax
from jax.experimental import pallas as pl
from jax.experimental.pallas import tpu as pltpu
from jax.experimental.pallas import tpu_sc as plsc

_VOCAB = 1000000
_EMBED = 100
_EPAD = 128
_B = 4096
_MAXLEN = 128
_N = _B * _MAXLEN
_ROW_W = _MAXLEN * _EMBED

_info = plsc.get_sparse_core_info()
_NC, _NS = _info.num_cores, _info.num_subcores
_NW = _NC * _NS
_SLAB = 8
_NSLAB = _B // (_SLAB * _NW)
_CH = 64
_KPS = _SLAB * _MAXLEN // _CH
_HALF = _CH * _EMBED

_PR = 200
_NPIECE = _VOCAB // _PR
_PMAX = -(-_NPIECE // 32)

_mesh = plsc.VectorSubcoreMesh(core_axis_name="c", subcore_axis_name="s")


@functools.partial(
    pl.kernel,
    mesh=_mesh,
    compiler_params=pltpu.CompilerParams(needs_layout_passes=False),
    out_type=jax.ShapeDtypeStruct((_VOCAB, _EPAD), jnp.float32),
    scratch_types=[
        pltpu.VMEM((2, _PR, _EMBED), jnp.float32),
        pltpu.VMEM((_PR, _EPAD), jnp.float32),
        pltpu.SemaphoreType.DMA,
        pltpu.SemaphoreType.DMA,
    ],
)
def _pad_sc(in_hbm, out_hbm, src_v, dst_v, isem0, isem1):
    wid = lax.axis_index("s") * _NC + lax.axis_index("c")
    iota16 = lax.iota(jnp.int32, 16)
    cvecs = [iota16 + jnp.int32(16 * g) for g in range(6)]
    cvec6 = jnp.where(iota16 >= 4, iota16 - 4, iota16 + 96)
    inc6 = (iota16 >= 4).astype(jnp.int32)
    zero, one = jnp.int32(0), jnp.int32(1)

    def in_cp(p, b, sem):
        return pltpu.make_async_copy(
            in_hbm.at[pl.ds((wid + 32 * p) * _PR, _PR), :],
            src_v.at[b],
            sem,
        )

    def expand_and_store(p, b):
        srcb = src_v.at[b]

        @plsc.parallel_loop(zero, jnp.int32(_PR), one, unroll=4)
        def expand(r):
            rvec = iota16 * 0 + r
            for g in range(6):
                dst_v[r, pl.ds(16 * g, 16)] = plsc.load_gather(
                    srcb, [rvec, cvecs[g]]
                )
            rv6 = jnp.minimum(rvec + inc6, jnp.int32(_PR - 1))
            dst_v[r, pl.ds(96, 16)] = plsc.load_gather(srcb, [rv6, cvec6])

        pltpu.sync_copy(dst_v, out_hbm.at[pl.ds((wid + 32 * p) * _PR, _PR)])

    @pl.when(wid < _NPIECE)
    def _():
        in_cp(zero, zero, isem0).start()

    def body(i, _):
        i = i.astype(jnp.int32)
        p0 = 2 * i
        p1 = 2 * i + 1

        @pl.when(wid + 32 * p1 < _NPIECE)
        def _():
            in_cp(p1, one, isem1).start()

        @pl.when(wid + 32 * p0 < _NPIECE)
        def _():
            in_cp(p0, zero, isem0).wait()

        @pl.when(wid + 32 * p0 < _NPIECE)
        def _():
            expand_and_store(p0, zero)

        @pl.when(wid + 32 * (p0 + 2) < _NPIECE)
        def _():
            in_cp(p0 + 2, zero, isem0).start()

        @pl.when(wid + 32 * p1 < _NPIECE)
        def _():
            in_cp(p1, one, isem1).wait()
            expand_and_store(p1, one)

        return _

    lax.fori_loop(zero, jnp.int32((_PMAX + 1) // 2), body, None)


@functools.partial(
    pl.kernel,
    mesh=_mesh,
    compiler_params=pltpu.CompilerParams(needs_layout_passes=False),
    out_type=jax.ShapeDtypeStruct((_B, _ROW_W), jnp.float32),
    scratch_types=[
        pltpu.VMEM((_KPS, _CH), jnp.int32),
        pltpu.VMEM((2, _CH, _EPAD), jnp.float32),
        pltpu.VMEM((_SLAB, _ROW_W), jnp.float32),
        pltpu.SemaphoreType.DMA,
        pltpu.SemaphoreType.DMA,
    ],
)
def _gather_kernel(table_hbm, idx_hbm, out_hbm,
                   idx_v, rows_v, cbuf, sem0, sem1):
    wid = lax.axis_index("s") * _NC + lax.axis_index("c")
    sems = (sem0, sem1)
    iota16 = lax.iota(jnp.int32, 16)

    def fire(k):
        return pltpu.async_copy(
            table_hbm.at[idx_v.at[jnp.int32(k)]],
            rows_v.at[jnp.int32(k & 1)],
            sems[k & 1],
        )

    def slab(s, _):
        s = s.astype(jnp.int32)
        q = wid * _NSLAB + s
        pltpu.sync_copy(idx_hbm.at[q], idx_v)
        cp = fire(0)
        for k in range(_KPS):
            nxt = fire(k + 1) if k + 1 < _KPS else None
            cp.wait()
            rr, h = jnp.int32(k >> 1), k & 1
            src = rows_v.at[jnp.int32(k & 1)]

            @plsc.parallel_loop(
                jnp.int32(0), jnp.int32(_HALF // 16), jnp.int32(1), unroll=8
            )
            def compact(j):
                c = iota16 + j * 16
                r = (c * 5243) >> 19
                e = c - r * _EMBED
                vals = plsc.load_gather(src, [r, e])
                cbuf[rr, pl.ds(h * _HALF + j * 16, 16)] = vals

            cp = nxt
        pltpu.sync_copy(cbuf, out_hbm.at[pl.ds(q * _SLAB, _SLAB)])
        return _

    lax.fori_loop(jnp.int32(0), jnp.int32(_NSLAB), slab, None)


def kernel(table, rev, lab):
    table_p = _pad_sc(table)
    idx = rev.astype(jnp.int32).reshape(_B // _SLAB, _KPS, _CH)
    out = _gather_kernel(table_p, idx)
    return (out, lab)


# speedup vs baseline: 1.0455x; 1.0455x over previous
import functools

import jax
import jax.numpy as jnp
from jax import lax
from jax.experimental import pallas as pl
from jax.experimental.pallas import tpu as pltpu
from jax.experimental.pallas import tpu_sc as plsc

_VOCAB = 1000000
_EMBED = 100
_EPAD = 128
_B = 4096
_MAXLEN = 128
_N = _B * _MAXLEN
_ROW_W = _MAXLEN * _EMBED

_info = plsc.get_sparse_core_info()
_NC, _NS = _info.num_cores, _info.num_subcores
_NW = _NC * _NS
_SLAB = 8
_NSLAB = _B // (_SLAB * _NW)
_CH = 64
_KPS = _SLAB * _MAXLEN // _CH
_HALF = _CH * _EMBED

_PAD_ROWS = 20000

_mesh = plsc.VectorSubcoreMesh(core_axis_name="c", subcore_axis_name="s")


def _pad_body(in_ref, out_ref):
    out_ref[:, :_EMBED] = in_ref[...]


_pad_tc = pl.pallas_call(
    _pad_body,
    grid=(_VOCAB // _PAD_ROWS,),
    in_specs=[pl.BlockSpec((_PAD_ROWS, _EMBED), lambda i: (i, jnp.int32(0)))],
    out_specs=pl.BlockSpec((_PAD_ROWS, _EPAD), lambda i: (i, jnp.int32(0))),
    out_shape=jax.ShapeDtypeStruct((_VOCAB, _EPAD), jnp.float32),
    compiler_params=pltpu.CompilerParams(dimension_semantics=("arbitrary",)),
)


@functools.partial(
    pl.kernel,
    mesh=_mesh,
    compiler_params=pltpu.CompilerParams(needs_layout_passes=False),
    out_type=jax.ShapeDtypeStruct((_B, _ROW_W), jnp.float32),
    scratch_types=[
        pltpu.VMEM((_KPS, _CH), jnp.int32),
        pltpu.VMEM((2, _CH, _EPAD), jnp.float32),
        pltpu.VMEM((_SLAB, _ROW_W), jnp.float32),
        pltpu.SemaphoreType.DMA,
        pltpu.SemaphoreType.DMA,
    ],
)
def _gather_kernel(table_hbm, idx_hbm, out_hbm,
                   idx_v, rows_v, cbuf, sem0, sem1):
    wid = lax.axis_index("s") * _NC + lax.axis_index("c")
    sems = (sem0, sem1)
    iota16 = lax.iota(jnp.int32, 16)

    def fire(k):
        return pltpu.async_copy(
            table_hbm.at[idx_v.at[jnp.int32(k)]],
            rows_v.at[jnp.int32(k & 1)],
            sems[k & 1],
        )

    def slab(s, _):
        s = s.astype(jnp.int32)
        q = wid * _NSLAB + s
        pltpu.sync_copy(idx_hbm.at[q], idx_v)
        cp = fire(0)
        for k in range(_KPS):
            nxt = fire(k + 1) if k + 1 < _KPS else None
            cp.wait()
            rr, h = jnp.int32(k >> 1), k & 1
            src = rows_v.at[jnp.int32(k & 1)]

            @plsc.parallel_loop(
                jnp.int32(0), jnp.int32(_HALF // 16), jnp.int32(1), unroll=8
            )
            def compact(j):
                c = iota16 + j * 16
                r = (c * 5243) >> 19
                e = c - r * _EMBED
                vals = plsc.load_gather(src, [r, e])
                cbuf[rr, pl.ds(h * _HALF + j * 16, 16)] = vals

            cp = nxt
        pltpu.sync_copy(cbuf, out_hbm.at[pl.ds(q * _SLAB, _SLAB)])
        return _

    lax.fori_loop(jnp.int32(0), jnp.int32(_NSLAB), slab, None)


def kernel(table, rev, lab):
    table_p = _pad_tc(table)
    idx = rev.astype(jnp.int32).reshape(_B // _SLAB, _KPS, _CH)
    out = _gather_kernel(table_p, idx)
    return (out, lab)


# speedup vs baseline: 1.0825x; 1.0354x over previous
import functools

import jax
import jax.numpy as jnp
from jax import lax
from jax.experimental import pallas as pl
from jax.experimental.pallas import tpu as pltpu
from jax.experimental.pallas import tpu_sc as plsc

_VOCAB = 1000000
_EMBED = 100
_EPAD = 128
_B = 4096
_MAXLEN = 128
_N = _B * _MAXLEN
_ROW_W = _MAXLEN * _EMBED

_info = plsc.get_sparse_core_info()
_NC, _NS = _info.num_cores, _info.num_subcores
_NW = _NC * _NS
_SLAB = 8
_NSLAB = _B // (_SLAB * _NW)
_CH = 64
_KPS = _SLAB * _MAXLEN // _CH
_HALF = _CH * _EMBED

_PAD_ROWS = 20000

_mesh = plsc.VectorSubcoreMesh(core_axis_name="c", subcore_axis_name="s")


def _pad_body(in_ref, out_ref):
    out_ref[:, :_EMBED] = in_ref[...]


_pad_tc = pl.pallas_call(
    _pad_body,
    grid=(_VOCAB // _PAD_ROWS,),
    in_specs=[pl.BlockSpec((_PAD_ROWS, _EMBED), lambda i: (i, jnp.int32(0)))],
    out_specs=pl.BlockSpec((_PAD_ROWS, _EPAD), lambda i: (i, jnp.int32(0))),
    out_shape=jax.ShapeDtypeStruct((_VOCAB, _EPAD), jnp.float32),
    compiler_params=pltpu.CompilerParams(dimension_semantics=("arbitrary",)),
)


@functools.partial(
    pl.kernel,
    mesh=_mesh,
    compiler_params=pltpu.CompilerParams(needs_layout_passes=False),
    out_type=jax.ShapeDtypeStruct((_B, _ROW_W), jnp.float32),
    scratch_types=[
        pltpu.VMEM((_KPS, _CH), jnp.int32),
        pltpu.VMEM((2, _CH, _EPAD), jnp.float32),
        pltpu.VMEM((_SLAB, _ROW_W), jnp.float32),
        pltpu.SemaphoreType.DMA,
        pltpu.SemaphoreType.DMA,
        pltpu.SemaphoreType.DMA,
        pltpu.SemaphoreType.DMA,
    ],
)
def _gather_kernel(table_hbm, idx_hbm, out_hbm,
                   idx_v, rows_v, cbuf, sem0, sem1, wsem0, wsem1):
    wid = lax.axis_index("s") * _NC + lax.axis_index("c")
    sems = (sem0, sem1)
    wsems = (wsem0, wsem1)
    iota16 = lax.iota(jnp.int32, 16)

    def fire(k):
        return pltpu.async_copy(
            table_hbm.at[idx_v.at[jnp.int32(k)]],
            rows_v.at[jnp.int32(k & 1)],
            sems[k & 1],
        )

    def wcopy(q, h):
        return pltpu.make_async_copy(
            cbuf.at[:, pl.ds(h * _HALF, _HALF)],
            out_hbm.at[pl.ds(q * _SLAB, _SLAB), pl.ds(h * _HALF, _HALF)],
            wsems[h],
        )

    def slab(s, _):
        s = s.astype(jnp.int32)
        q = wid * _NSLAB + s
        pltpu.sync_copy(idx_hbm.at[q], idx_v)
        cp = fire(0)
        for k in range(_KPS):
            nxt = fire(k + 1) if k + 1 < _KPS else None
            cp.wait()
            rr, h = jnp.int32(k & 7), k >> 3
            src = rows_v.at[jnp.int32(k & 1)]
            if k == 0 or k == _KPS // 2:
                @pl.when(s > 0)
                def _():
                    wcopy(q, h).wait()

            @plsc.parallel_loop(
                jnp.int32(0), jnp.int32(_HALF // 16), jnp.int32(1), unroll=8
            )
            def compact(j):
                c = iota16 + j * 16
                r = (c * 5243) >> 19
                e = c - r * _EMBED
                vals = plsc.load_gather(src, [r, e])
                cbuf[rr, pl.ds(h * _HALF + j * 16, 16)] = vals

            if k == _KPS // 2 - 1 or k == _KPS - 1:
                wcopy(q, h).start()
            cp = nxt
        return _

    lax.fori_loop(jnp.int32(0), jnp.int32(_NSLAB), slab, None)
    last_q = wid * _NSLAB + jnp.int32(_NSLAB - 1)
    wcopy(last_q, 0).wait()
    wcopy(last_q, 1).wait()


def kernel(table, rev, lab):
    table_p = _pad_tc(table)
    idx = rev.astype(jnp.int32).reshape(_B // _SLAB, _SLAB, 2, _CH)
    idx = idx.transpose(0, 2, 1, 3).reshape(_B // _SLAB, _KPS, _CH)
    out = _gather_kernel(table_p, idx)
    return (out, lab)
